# baseline (speedup 1.0000x reference)
import functools

import jax
import jax.numpy as jnp
import numpy as np
from jax import lax
from jax.experimental import pallas as pl
from jax.experimental.pallas import tpu as pltpu
from jax.experimental.pallas import tpu_sc as plsc

N = 10000
E = 320000
D = 128
G = 16
CONV_CFG = [(D, 8, 32, True), (256, 8, 32, True), (256, 8, 32, True), (256, 1, 256, False)]

ETOT = E + N
NW = 32
NB = 64
R = 160
NP = NB * R
BE = 16
ALIGN = 256
SCB = 16
EP = 346624
NBLK = EP // BE
EA = 320008


def _mm_kernel(a_ref, b_ref, o_ref):
    o_ref[...] = jnp.dot(a_ref[...], b_ref[...], preferred_element_type=jnp.float32)


def _mm(a, b, bm):
    m, k = a.shape
    _, n = b.shape
    return pl.pallas_call(
        _mm_kernel,
        grid=(m // bm,),
        in_specs=[
            pl.BlockSpec((bm, k), lambda i: (i, 0)),
            pl.BlockSpec((k, n), lambda i: (0, 0)),
        ],
        out_specs=pl.BlockSpec((bm, n), lambda i: (i, 0)),
        out_shape=jax.ShapeDtypeStruct((m, n), jnp.float32),
    )(a, b)


def _mm_ae_kernel(a_ref, b_ref, o_ref, s_ref):
    i = pl.program_id(0)
    o = jnp.dot(a_ref[...], b_ref[...], preferred_element_type=jnp.float32)
    o_ref[...] = o

    @pl.when(i == 0)
    def _():
        s_ref[...] = jnp.zeros_like(s_ref)

    s_ref[...] += jnp.broadcast_to(jnp.sum(o, axis=0, keepdims=True), s_ref.shape)


def _mm_ae(a, b):
    m, k = a.shape
    _, n = b.shape
    bm = 1000
    return pl.pallas_call(
        _mm_ae_kernel,
        grid=(m // bm,),
        in_specs=[
            pl.BlockSpec((bm, k), lambda i: (i, 0)),
            pl.BlockSpec((k, n), lambda i: (0, 0)),
        ],
        out_specs=[
            pl.BlockSpec((bm, n), lambda i: (i, 0)),
            pl.BlockSpec((8, n), lambda i: (0, 0)),
        ],
        out_shape=[
            jax.ShapeDtypeStruct((m, n), jnp.float32),
            jax.ShapeDtypeStruct((8, n), jnp.float32),
        ],
    )(a, b)


def _post_kernel(o_ref, h_ref, sw_ref, v_ref, out_ref):
    o = o_ref[...]
    v = v_ref[...]
    o = (o + v[0:1]) * v[1:2] + v[2:3]
    mu = jnp.mean(o, axis=1, keepdims=True)
    var = jnp.mean((o - mu) ** 2, axis=1, keepdims=True)
    o = (o - mu) / jnp.sqrt(var + 1e-5) * v[3:4] + v[4:5]
    o = o + jnp.dot(h_ref[...], sw_ref[...], preferred_element_type=jnp.float32) + v[5:6]
    out_ref[...] = jnp.where(o > 0, o, jnp.exp(o) - 1.0)


def _post(o, h, sw, vecs):
    m = o.shape[0]
    cin = h.shape[1]
    bm = 400
    return pl.pallas_call(
        _post_kernel,
        grid=(m // bm,),
        in_specs=[
            pl.BlockSpec((bm, 256), lambda i: (i, 0)),
            pl.BlockSpec((bm, cin), lambda i: (i, 0)),
            pl.BlockSpec((cin, 256), lambda i: (0, 0)),
            pl.BlockSpec((8, 256), lambda i: (0, 0)),
        ],
        out_specs=pl.BlockSpec((bm, 256), lambda i: (i, 0)),
        out_shape=jax.ShapeDtypeStruct((m, 256), jnp.float32),
    )(o, h, sw, vecs)


def _pool_kernel(b_ref, h_ref, out_ref):
    i = pl.program_id(0)

    @pl.when(i == 0)
    def _():
        out_ref[...] = jnp.zeros_like(out_ref)

    b = b_ref[0, 0, :]
    oh = (b[:, None] == lax.broadcasted_iota(jnp.int32, (1, G), 1)).astype(jnp.float32)
    hb = jnp.concatenate(
        [h_ref[...], jnp.ones((h_ref.shape[0], 128), jnp.float32)], axis=1)
    out_ref[...] += lax.dot_general(
        oh, hb, (((0,), (0,)), ((), ())), preferred_element_type=jnp.float32)

    @pl.when(i == pl.num_programs(0) - 1)
    def _():
        acc = out_ref[...]
        out_ref[...] = acc / jnp.maximum(acc[:, 256:257], 1.0)


def _pool(batch3, h):
    bm = 1000
    return pl.pallas_call(
        _pool_kernel,
        grid=(N // bm,),
        in_specs=[
            pl.BlockSpec((1, 1, bm), lambda i: (i, 0, 0)),
            pl.BlockSpec((bm, 256), lambda i: (i, 0)),
        ],
        out_specs=pl.BlockSpec((G, 384), lambda i: (0, 0)),
        out_shape=jax.ShapeDtypeStruct((G, 384), jnp.float32),
    )(batch3, h)


def _make_sc_gat(C, LOFF):
    mesh = plsc.VectorSubcoreMesh(core_axis_name="c", subcore_axis_name="s")
    NC = 2

    @functools.partial(
        pl.kernel,
        out_type=jax.ShapeDtypeStruct((NP, 256), jnp.float32),
        mesh=mesh,
        scratch_types=[
            pltpu.VMEM((R, 256), jnp.float32),
            pltpu.VMEM((40, 128), jnp.float32),
            pltpu.VMEM((R, 16), jnp.float32),
            pltpu.VMEM((3 * SCB * BE + 16,), jnp.int32),
            pltpu.VMEM((BE, 128), jnp.float32),
            pltpu.VMEM((BE, 128), jnp.float32),
            pltpu.VMEM((BE, 128), jnp.float32),
            pltpu.VMEM((BE, 128), jnp.float32),
            pltpu.VMEM((BE, 256), jnp.float32),
            pltpu.VMEM((BE, 256), jnp.float32),
            pltpu.SemaphoreType.DMA,
            pltpu.SemaphoreType.DMA,
        ],
    )
    def k(xw, asrc, adstc, aedge, src2, dstl2, eid2, startsb, out,
          acc, adst_loc, rden, idx_c,
          asrc_g0, asrc_g1, aedge_g0, aedge_g1, xw_g0, xw_g1,
          sem0, sem1):
        SB = SCB * BE
        src_c = idx_c.at[pl.ds(0, SB)]
        eid_c = idx_c.at[pl.ds(2 * SB, SB)]
        w = lax.axis_index("s") * NC + lax.axis_index("c")

        def adst_body(ac, _c):
            ao = pl.multiple_of(ac * 8, 8)
            pltpu.sync_copy(adstc.at[pl.ds(w * 40 + ao, 8)],
                            adst_loc.at[pl.ds(ao, 8)])
            return 0

        lax.fori_loop(0, 5, adst_body, 0)

        def half_body(half, _h):
            bkt = w * 2 + half
            rlo = bkt * R
            half20 = half * 20
            pltpu.sync_copy(startsb, idx_c.at[pl.ds(0, 80)])
            bvec = idx_c[pl.ds(bkt, 16)]
            b0 = bvec[0]
            b1 = bvec[1]

            def zero_body(n, _):
                zz = jnp.zeros((16,), jnp.float32)
                for kk in range(16):
                    acc[n, pl.ds(kk * 16, 16)] = zz
                rden[n, :] = zz
                return 0

            lax.fori_loop(0, R, zero_body, 0)

            def run(pass2):
                gbufs = [(asrc_g0, aedge_g0, xw_g0, sem0),
                         (asrc_g1, aedge_g1, xw_g1, sem1)]

                def issue(j, bufs):
                    a_g, e_g, x_g, sem = bufs
                    sidx = src_c.at[pl.ds(j * BE, BE)]
                    eidx = eid_c.at[pl.ds(j * BE, BE)]
                    pltpu.async_copy(asrc.at[sidx], a_g, sem)
                    pltpu.async_copy(aedge.at[eidx], e_g, sem)
                    if pass2:
                        pltpu.async_copy(xw.at[sidx], x_g, sem)

                def drain(bufs):
                    a_g, e_g, x_g, sem = bufs
                    didx = src_c.at[pl.ds(0, BE)]
                    pltpu.make_async_copy(asrc.at[didx], a_g, sem).wait()
                    pltpu.make_async_copy(aedge.at[didx], e_g, sem).wait()
                    if pass2:
                        pltpu.make_async_copy(xw.at[didx], x_g, sem).wait()

                def compute(j, bufs):
                    a_g, e_g, x_g, _ = bufs
                    dvec = idx_c[pl.ds(SB + j * BE, 16)]
                    for t in range(BE):
                        dstl = dvec[t]
                        al = (a_g[t, pl.ds(0, 16)] + e_g[t, pl.ds(LOFF, 16)]
                              + adst_loc[half20 + dstl // 8,
                                         pl.ds((dstl % 8) * 16, 16)])
                        al = jnp.maximum(al, al * 0.2)
                        ex = jnp.exp(al)
                        if not pass2:
                            plsc.addupdate(rden.at[dstl], ex)
                        else:
                            att = ex * rden[dstl, :]
                            for kk in range(16):
                                hk = (kk * 16) // C
                                plsc.addupdate(
                                    acc.at[dstl, pl.ds(kk * 16, 16)],
                                    x_g[t, pl.ds(kk * 16, 16)] * att[hk])

                def sc_body(sci, _c):
                    bs = pl.multiple_of(b0 + sci * SCB, 8)
                    cnt = jnp.minimum(b1 - bs, SCB)
                    be = pl.multiple_of(bs * BE, 128)
                    pltpu.sync_copy(src2.at[pl.ds(be, SB)],
                                    src_c.at[pl.ds(0, SB)])
                    pltpu.sync_copy(dstl2.at[pl.ds(be, SB)],
                                    idx_c.at[pl.ds(SB, SB)])
                    pltpu.sync_copy(eid2.at[pl.ds(be, SB)],
                                    eid_c.at[pl.ds(0, SB)])
                    issue(0, gbufs[0])

                    def blk(j, _b):
                        even = (j % 2) == 0

                        @pl.when((j + 1 < cnt) & even)
                        def _():
                            issue(j + 1, gbufs[1])

                        @pl.when((j + 1 < cnt) & jnp.logical_not(even))
                        def _():
                            issue(j + 1, gbufs[0])

                        @pl.when(even)
                        def _():
                            drain(gbufs[0])
                            compute(j, gbufs[0])

                        @pl.when(jnp.logical_not(even))
                        def _():
                            drain(gbufs[1])
                            compute(j, gbufs[1])

                        return 0

                    lax.fori_loop(0, cnt, blk, 0)
                    return 0

                nsc = (b1 - b0 + SCB - 1) // SCB
                lax.fori_loop(0, nsc, sc_body, 0)

            run(False)

            def rcp_body(n, _c):
                rden[n, :] = 1.0 / (rden[n, :] + 1e-16)
                return 0

            lax.fori_loop(0, R, rcp_body, 0)
            run(True)

            def out_body(oc, _c):
                oo = pl.multiple_of(rlo + oc * 8, 8)
                ol = pl.multiple_of(oc * 8, 8)
                pltpu.sync_copy(acc.at[pl.ds(ol, 8), pl.ds(0, 128)],
                                out.at[pl.ds(oo, 8), pl.ds(0, 128)])
                pltpu.sync_copy(acc.at[pl.ds(ol, 8), pl.ds(128, 128)],
                                out.at[pl.ds(oo, 8), pl.ds(128, 128)])
                return 0

            lax.fori_loop(0, 20, out_body, 0)
            return 0

        lax.fori_loop(0, 2, half_body, 0)

    return k


_sc_gat = [_make_sc_gat(32, 0), _make_sc_gat(32, 16),
           _make_sc_gat(32, 32), _make_sc_gat(256, 48)]


def kernel(x, edge_index, edge_attr, batch, params):
    wcat, skw, vecs, cdim = [], [], [], []
    for i, (cin, h, c, concat) in enumerate(CONV_CFG):
        lw = params[f'conv{i}_lin_w']
        bsrc = (lw.reshape(h, c, cin) * params[f'conv{i}_att_src'][:, :, None]).sum(1).T
        bdst = (lw.reshape(h, c, cin) * params[f'conv{i}_att_dst'][:, :, None]).sum(1).T
        w = jnp.zeros((cin, 512), jnp.float32)
        w = w.at[:, :256].set(lw.T)
        w = w.at[:, 256:256 + h].set(bsrc)
        w = w.at[:, 272:272 + h].set(bdst)
        wcat.append(w)
        if i > 0:
            skw.append(params[f'skip{i}_w'].T)
            skb = params[f'skip{i}_b']
        else:
            skw.append(jnp.zeros((cin, 256), jnp.float32))
            skb = jnp.zeros((256,), jnp.float32)
        bn_scale = params[f'bn{i}_g'] / np.sqrt(1.0 + 1e-5)
        v = jnp.stack([params[f'conv{i}_bias'], bn_scale, params[f'bn{i}_b'],
                       params[f'ln{i}_g'], params[f'ln{i}_b'], skb,
                       jnp.zeros((256,), jnp.float32), jnp.zeros((256,), jnp.float32)])
        vecs.append(v)
        cdim.append(c)

    bedge = jnp.zeros((D, 64), jnp.float32)
    for i, (cin, h, c, concat) in enumerate(CONV_CFG):
        lew = params[f'conv{i}_lin_edge_w']
        aev = params[f'conv{i}_att_edge']
        col = (lew.reshape(h, c, D) * aev[:, :, None]).sum(1).T
        bedge = bedge.at[:, 16 * i:16 * i + h].set(col)

    loop_idx = jnp.arange(N, dtype=jnp.int32)
    src = jnp.concatenate([edge_index[0], loop_idx])
    dst = jnp.concatenate([edge_index[1], loop_idx])
    eid = jnp.concatenate([jnp.arange(E, dtype=jnp.int32),
                           jnp.full((N,), E, jnp.int32)])
    order = jnp.argsort(dst)
    src_s, dst_s, eid_s = src[order], dst[order], eid[order]
    bucket_s = dst_s // R
    bnd = jnp.searchsorted(dst_s, jnp.arange(NB + 1, dtype=jnp.int32) * R).astype(jnp.int32)
    sizes = jnp.diff(bnd)
    astart = jnp.concatenate([
        jnp.zeros((1,), jnp.int32),
        jnp.cumsum(((sizes + ALIGN - 1) // ALIGN) * ALIGN).astype(jnp.int32)])
    pos = astart[bucket_s] + (jnp.arange(ETOT, dtype=jnp.int32) - bnd[bucket_s])
    fill = jnp.broadcast_to(jnp.array([0, 0, E + 1], jnp.int32), (EP, 3))
    packed = fill.at[pos].set(
        jnp.stack([src_s, dst_s - bucket_s * R, eid_s], axis=1))
    src2 = packed[:, 0]
    dstl2 = packed[:, 1]
    eid2 = packed[:, 2]
    startsb = jnp.pad(astart // BE, (0, 15))

    ae_all, ae_sum = _mm_ae(edge_attr, bedge)
    ae_mean = ae_sum[0:1] / E
    aedge_cmb = jnp.zeros((EA, 128), jnp.float32)
    aedge_cmb = aedge_cmb.at[:E, :64].set(ae_all)
    aedge_cmb = aedge_cmb.at[E:E + 1, :64].set(ae_mean)
    aedge_cmb = aedge_cmb.at[E + 1, :].set(-1e30)

    h_cur = x
    for i, (cin, h, c, concat) in enumerate(CONV_CFG):
        cat = _mm(h_cur, wcat[i], 400)
        xw = cat[:, :256]
        ad_all = jnp.pad(cat[:, 256:384], ((0, NP - N), (0, 0)))
        adst_cmp = jnp.pad(cat[:, 272:288], ((0, NP - N), (0, 0))).reshape(NP // 8, 128)
        out_sc = _sc_gat[i](xw, ad_all, adst_cmp, aedge_cmb,
                            src2, dstl2, eid2, startsb)[:N]
        h_cur = _post(out_sc, h_cur, skw[i], vecs[i])

    batch3 = batch.reshape(10, 1, 1000)
    return _pool(batch3, h_cur)[:, :256]


# speedup vs baseline: 1.0311x; 1.0311x over previous
import functools

import jax
import jax.numpy as jnp
import numpy as np
from jax import lax
from jax.experimental import pallas as pl
from jax.experimental.pallas import tpu as pltpu
from jax.experimental.pallas import tpu_sc as plsc

N = 10000
E = 320000
D = 128
G = 16
CONV_CFG = [(D, 8, 32, True), (256, 8, 32, True), (256, 8, 32, True), (256, 1, 256, False)]

ETOT = E + N
NW = 32
NB = 64
R = 160
NP = NB * R
BE = 8
ALIGN = 256
SCB = 32
EP = 346624
NBLK = EP // BE
EA = 320008


def _mm_kernel(a_ref, b_ref, o_ref):
    o_ref[...] = jnp.dot(a_ref[...], b_ref[...], preferred_element_type=jnp.float32)


def _mm(a, b, bm):
    m, k = a.shape
    _, n = b.shape
    return pl.pallas_call(
        _mm_kernel,
        grid=(m // bm,),
        in_specs=[
            pl.BlockSpec((bm, k), lambda i: (i, 0)),
            pl.BlockSpec((k, n), lambda i: (0, 0)),
        ],
        out_specs=pl.BlockSpec((bm, n), lambda i: (i, 0)),
        out_shape=jax.ShapeDtypeStruct((m, n), jnp.float32),
    )(a, b)


def _mm_ae_kernel(a_ref, b_ref, o_ref, s_ref):
    i = pl.program_id(0)
    o = jnp.dot(a_ref[...], b_ref[...], preferred_element_type=jnp.float32)
    o_ref[...] = o

    @pl.when(i == 0)
    def _():
        s_ref[...] = jnp.zeros_like(s_ref)

    s_ref[...] += jnp.broadcast_to(jnp.sum(o, axis=0, keepdims=True), s_ref.shape)


def _mm_ae(a, b):
    m, k = a.shape
    _, n = b.shape
    bm = 1000
    return pl.pallas_call(
        _mm_ae_kernel,
        grid=(m // bm,),
        in_specs=[
            pl.BlockSpec((bm, k), lambda i: (i, 0)),
            pl.BlockSpec((k, n), lambda i: (0, 0)),
        ],
        out_specs=[
            pl.BlockSpec((bm, n), lambda i: (i, 0)),
            pl.BlockSpec((8, n), lambda i: (0, 0)),
        ],
        out_shape=[
            jax.ShapeDtypeStruct((m, n), jnp.float32),
            jax.ShapeDtypeStruct((8, n), jnp.float32),
        ],
    )(a, b)


def _post_kernel(o_ref, h_ref, sw_ref, v_ref, out_ref):
    o = o_ref[...]
    v = v_ref[...]
    o = (o + v[0:1]) * v[1:2] + v[2:3]
    mu = jnp.mean(o, axis=1, keepdims=True)
    var = jnp.mean((o - mu) ** 2, axis=1, keepdims=True)
    o = (o - mu) / jnp.sqrt(var + 1e-5) * v[3:4] + v[4:5]
    o = o + jnp.dot(h_ref[...], sw_ref[...], preferred_element_type=jnp.float32) + v[5:6]
    out_ref[...] = jnp.where(o > 0, o, jnp.exp(o) - 1.0)


def _post(o, h, sw, vecs):
    m = o.shape[0]
    cin = h.shape[1]
    bm = 400
    return pl.pallas_call(
        _post_kernel,
        grid=(m // bm,),
        in_specs=[
            pl.BlockSpec((bm, 256), lambda i: (i, 0)),
            pl.BlockSpec((bm, cin), lambda i: (i, 0)),
            pl.BlockSpec((cin, 256), lambda i: (0, 0)),
            pl.BlockSpec((8, 256), lambda i: (0, 0)),
        ],
        out_specs=pl.BlockSpec((bm, 256), lambda i: (i, 0)),
        out_shape=jax.ShapeDtypeStruct((m, 256), jnp.float32),
    )(o, h, sw, vecs)


def _pool_kernel(b_ref, h_ref, out_ref):
    i = pl.program_id(0)

    @pl.when(i == 0)
    def _():
        out_ref[...] = jnp.zeros_like(out_ref)

    b = b_ref[0, 0, :]
    oh = (b[:, None] == lax.broadcasted_iota(jnp.int32, (1, G), 1)).astype(jnp.float32)
    hb = jnp.concatenate(
        [h_ref[...], jnp.ones((h_ref.shape[0], 128), jnp.float32)], axis=1)
    out_ref[...] += lax.dot_general(
        oh, hb, (((0,), (0,)), ((), ())), preferred_element_type=jnp.float32)

    @pl.when(i == pl.num_programs(0) - 1)
    def _():
        acc = out_ref[...]
        out_ref[...] = acc / jnp.maximum(acc[:, 256:257], 1.0)


def _pool(batch3, h):
    bm = 1000
    return pl.pallas_call(
        _pool_kernel,
        grid=(N // bm,),
        in_specs=[
            pl.BlockSpec((1, 1, bm), lambda i: (i, 0, 0)),
            pl.BlockSpec((bm, 256), lambda i: (i, 0)),
        ],
        out_specs=pl.BlockSpec((G, 384), lambda i: (0, 0)),
        out_shape=jax.ShapeDtypeStruct((G, 384), jnp.float32),
    )(batch3, h)


def _make_sc_gat(C, LOFF):
    mesh = plsc.VectorSubcoreMesh(core_axis_name="c", subcore_axis_name="s")
    NC = 2

    @functools.partial(
        pl.kernel,
        out_type=jax.ShapeDtypeStruct((NP, 256), jnp.float32),
        mesh=mesh,
        scratch_types=[
            pltpu.VMEM((R, 256), jnp.float32),
            pltpu.VMEM((40, 128), jnp.float32),
            pltpu.VMEM((R, 16), jnp.float32),
            pltpu.VMEM((3 * SCB * BE + 16,), jnp.int32),
            pltpu.VMEM((BE, 128), jnp.float32),
            pltpu.VMEM((BE, 128), jnp.float32),
            pltpu.VMEM((BE, 128), jnp.float32),
            pltpu.VMEM((BE, 128), jnp.float32),
            pltpu.VMEM((BE, 256), jnp.float32),
            pltpu.VMEM((BE, 256), jnp.float32),
            pltpu.SemaphoreType.DMA,
            pltpu.SemaphoreType.DMA,
        ],
    )
    def k(xw, asrc, adstc, aedge, src2, dstl2, eid2, startsb, out,
          acc, adst_loc, rden, idx_c,
          asrc_g0, asrc_g1, aedge_g0, aedge_g1, xw_g0, xw_g1,
          sem0, sem1):
        SB = SCB * BE
        src_c = idx_c.at[pl.ds(0, SB)]
        eid_c = idx_c.at[pl.ds(2 * SB, SB)]
        w = lax.axis_index("s") * NC + lax.axis_index("c")

        def adst_body(ac, _c):
            ao = pl.multiple_of(ac * 8, 8)
            pltpu.sync_copy(adstc.at[pl.ds(w * 40 + ao, 8)],
                            adst_loc.at[pl.ds(ao, 8)])
            return 0

        lax.fori_loop(0, 5, adst_body, 0)

        def half_body(half, _h):
            bkt = w * 2 + half
            rlo = bkt * R
            half20 = half * 20
            pltpu.sync_copy(startsb, idx_c.at[pl.ds(0, 80)])
            bvec = idx_c[pl.ds(bkt, 16)]
            b0 = bvec[0]
            b1 = bvec[1]

            def zero_body(n, _):
                zz = jnp.zeros((16,), jnp.float32)
                for kk in range(16):
                    acc[n, pl.ds(kk * 16, 16)] = zz
                rden[n, :] = zz
                return 0

            lax.fori_loop(0, R, zero_body, 0)

            def run(pass2):
                gbufs = [(asrc_g0, aedge_g0, xw_g0, sem0),
                         (asrc_g1, aedge_g1, xw_g1, sem1)]

                def issue(j, bufs):
                    a_g, e_g, x_g, sem = bufs
                    sidx = src_c.at[pl.ds(j * BE, BE)]
                    eidx = eid_c.at[pl.ds(j * BE, BE)]
                    pltpu.async_copy(asrc.at[sidx], a_g, sem)
                    pltpu.async_copy(aedge.at[eidx], e_g, sem)
                    if pass2:
                        pltpu.async_copy(xw.at[sidx], x_g, sem)

                def drain(bufs):
                    a_g, e_g, x_g, sem = bufs
                    didx = src_c.at[pl.ds(0, BE)]
                    pltpu.make_async_copy(asrc.at[didx], a_g, sem).wait()
                    pltpu.make_async_copy(aedge.at[didx], e_g, sem).wait()
                    if pass2:
                        pltpu.make_async_copy(xw.at[didx], x_g, sem).wait()

                def compute(j, bufs):
                    a_g, e_g, x_g, _ = bufs
                    dvec = idx_c[pl.ds(SB + j * BE, 16)]
                    for t in range(BE):
                        dstl = dvec[t]
                        al = (a_g[t, pl.ds(0, 16)] + e_g[t, pl.ds(LOFF, 16)]
                              + adst_loc[half20 + dstl // 8,
                                         pl.ds((dstl % 8) * 16, 16)])
                        al = jnp.maximum(al, al * 0.2)
                        ex = jnp.exp(al)
                        if not pass2:
                            plsc.addupdate(rden.at[dstl], ex)
                        else:
                            att = ex * rden[dstl, :]
                            for kk in range(16):
                                hk = (kk * 16) // C
                                plsc.addupdate(
                                    acc.at[dstl, pl.ds(kk * 16, 16)],
                                    x_g[t, pl.ds(kk * 16, 16)] * att[hk])

                def sc_body(sci, _c):
                    bs = pl.multiple_of(b0 + sci * SCB, 8)
                    cnt = jnp.minimum(b1 - bs, SCB)
                    be = pl.multiple_of(bs * BE, 128)
                    pltpu.sync_copy(src2.at[pl.ds(be, SB)],
                                    src_c.at[pl.ds(0, SB)])
                    pltpu.sync_copy(dstl2.at[pl.ds(be, SB)],
                                    idx_c.at[pl.ds(SB, SB)])
                    pltpu.sync_copy(eid2.at[pl.ds(be, SB)],
                                    eid_c.at[pl.ds(0, SB)])
                    issue(0, gbufs[0])

                    def blk(j, _b):
                        even = (j % 2) == 0

                        @pl.when((j + 1 < cnt) & even)
                        def _():
                            issue(j + 1, gbufs[1])

                        @pl.when((j + 1 < cnt) & jnp.logical_not(even))
                        def _():
                            issue(j + 1, gbufs[0])

                        @pl.when(even)
                        def _():
                            drain(gbufs[0])
                            compute(j, gbufs[0])

                        @pl.when(jnp.logical_not(even))
                        def _():
                            drain(gbufs[1])
                            compute(j, gbufs[1])

                        return 0

                    lax.fori_loop(0, cnt, blk, 0)
                    return 0

                nsc = (b1 - b0 + SCB - 1) // SCB
                lax.fori_loop(0, nsc, sc_body, 0)

            run(False)

            def rcp_body(n, _c):
                rden[n, :] = 1.0 / (rden[n, :] + 1e-16)
                return 0

            lax.fori_loop(0, R, rcp_body, 0)
            run(True)

            def out_body(oc, _c):
                oo = pl.multiple_of(rlo + oc * 8, 8)
                ol = pl.multiple_of(oc * 8, 8)
                pltpu.sync_copy(acc.at[pl.ds(ol, 8), pl.ds(0, 128)],
                                out.at[pl.ds(oo, 8), pl.ds(0, 128)])
                pltpu.sync_copy(acc.at[pl.ds(ol, 8), pl.ds(128, 128)],
                                out.at[pl.ds(oo, 8), pl.ds(128, 128)])
                return 0

            lax.fori_loop(0, 20, out_body, 0)
            return 0

        lax.fori_loop(0, 2, half_body, 0)

    return k


_sc_gat = [_make_sc_gat(32, 0), _make_sc_gat(32, 16),
           _make_sc_gat(32, 32), _make_sc_gat(256, 48)]


def kernel(x, edge_index, edge_attr, batch, params):
    wcat, skw, vecs, cdim = [], [], [], []
    for i, (cin, h, c, concat) in enumerate(CONV_CFG):
        lw = params[f'conv{i}_lin_w']
        bsrc = (lw.reshape(h, c, cin) * params[f'conv{i}_att_src'][:, :, None]).sum(1).T
        bdst = (lw.reshape(h, c, cin) * params[f'conv{i}_att_dst'][:, :, None]).sum(1).T
        w = jnp.zeros((cin, 512), jnp.float32)
        w = w.at[:, :256].set(lw.T)
        w = w.at[:, 256:256 + h].set(bsrc)
        w = w.at[:, 272:272 + h].set(bdst)
        wcat.append(w)
        if i > 0:
            skw.append(params[f'skip{i}_w'].T)
            skb = params[f'skip{i}_b']
        else:
            skw.append(jnp.zeros((cin, 256), jnp.float32))
            skb = jnp.zeros((256,), jnp.float32)
        bn_scale = params[f'bn{i}_g'] / np.sqrt(1.0 + 1e-5)
        v = jnp.stack([params[f'conv{i}_bias'], bn_scale, params[f'bn{i}_b'],
                       params[f'ln{i}_g'], params[f'ln{i}_b'], skb,
                       jnp.zeros((256,), jnp.float32), jnp.zeros((256,), jnp.float32)])
        vecs.append(v)
        cdim.append(c)

    bedge = jnp.zeros((D, 64), jnp.float32)
    for i, (cin, h, c, concat) in enumerate(CONV_CFG):
        lew = params[f'conv{i}_lin_edge_w']
        aev = params[f'conv{i}_att_edge']
        col = (lew.reshape(h, c, D) * aev[:, :, None]).sum(1).T
        bedge = bedge.at[:, 16 * i:16 * i + h].set(col)

    loop_idx = jnp.arange(N, dtype=jnp.int32)
    src = jnp.concatenate([edge_index[0], loop_idx])
    dst = jnp.concatenate([edge_index[1], loop_idx])
    eid = jnp.concatenate([jnp.arange(E, dtype=jnp.int32),
                           jnp.full((N,), E, jnp.int32)])
    dst_s, src_s, eid_s = lax.sort((dst, src, eid), num_keys=1)
    bucket_s = dst_s // R
    bnd = jnp.searchsorted(dst_s, jnp.arange(NB + 1, dtype=jnp.int32) * R).astype(jnp.int32)
    sizes = jnp.diff(bnd)
    astart = jnp.concatenate([
        jnp.zeros((1,), jnp.int32),
        jnp.cumsum(((sizes + ALIGN - 1) // ALIGN) * ALIGN).astype(jnp.int32)])
    pos = astart[bucket_s] + (jnp.arange(ETOT, dtype=jnp.int32) - bnd[bucket_s])
    src2 = jnp.zeros((EP,), jnp.int32).at[pos].set(
        src_s, mode='promise_in_bounds', unique_indices=True)
    dstl2 = jnp.zeros((EP,), jnp.int32).at[pos].set(
        dst_s - bucket_s * R, mode='promise_in_bounds', unique_indices=True)
    eid2 = jnp.full((EP,), E + 1, jnp.int32).at[pos].set(
        eid_s, mode='promise_in_bounds', unique_indices=True)
    startsb = jnp.pad(astart // BE, (0, 15))

    ae_all, ae_sum = _mm_ae(edge_attr, bedge)
    ae_mean = ae_sum[0:1] / E
    aedge_cmb = jnp.zeros((EA, 128), jnp.float32)
    aedge_cmb = aedge_cmb.at[:E, :64].set(ae_all)
    aedge_cmb = aedge_cmb.at[E:E + 1, :64].set(ae_mean)
    aedge_cmb = aedge_cmb.at[E + 1, :].set(-1e30)

    h_cur = x
    for i, (cin, h, c, concat) in enumerate(CONV_CFG):
        cat = _mm(h_cur, wcat[i], 400)
        xw = cat[:, :256]
        ad_all = jnp.pad(cat[:, 256:384], ((0, NP - N), (0, 0)))
        adst_cmp = jnp.pad(cat[:, 272:288], ((0, NP - N), (0, 0))).reshape(NP // 8, 128)
        out_sc = _sc_gat[i](xw, ad_all, adst_cmp, aedge_cmb,
                            src2, dstl2, eid2, startsb)[:N]
        h_cur = _post(out_sc, h_cur, skw[i], vecs[i])

    batch3 = batch.reshape(10, 1, 1000)
    return _pool(batch3, h_cur)[:, :256]


# speedup vs baseline: 1.0581x; 1.0262x over previous
import functools

import jax
import jax.numpy as jnp
import numpy as np
from jax import lax
from jax.experimental import pallas as pl
from jax.experimental.pallas import tpu as pltpu
from jax.experimental.pallas import tpu_sc as plsc

N = 10000
E = 320000
D = 128
G = 16
CONV_CFG = [(D, 8, 32, True), (256, 8, 32, True), (256, 8, 32, True), (256, 1, 256, False)]

ETOT = E + N
NW = 32
NB = 64
R = 160
NP = NB * R
BE = 8
ALIGN = 256
SCB = 32
EP = 346624
NBLK = EP // BE
EA = 320008


def _mm_kernel(a_ref, b_ref, o_ref):
    o_ref[...] = jnp.dot(a_ref[...], b_ref[...], preferred_element_type=jnp.float32)


def _mm(a, b, bm):
    m, k = a.shape
    _, n = b.shape
    return pl.pallas_call(
        _mm_kernel,
        grid=(m // bm,),
        in_specs=[
            pl.BlockSpec((bm, k), lambda i: (i, 0)),
            pl.BlockSpec((k, n), lambda i: (0, 0)),
        ],
        out_specs=pl.BlockSpec((bm, n), lambda i: (i, 0)),
        out_shape=jax.ShapeDtypeStruct((m, n), jnp.float32),
    )(a, b)


def _mm_ae_kernel(a_ref, b_ref, o_ref, s_ref):
    i = pl.program_id(0)
    o = jnp.dot(a_ref[...], b_ref[...], preferred_element_type=jnp.float32)
    o_ref[...] = o

    @pl.when(i == 0)
    def _():
        s_ref[...] = jnp.zeros_like(s_ref)

    s_ref[...] += jnp.broadcast_to(jnp.sum(o, axis=0, keepdims=True), s_ref.shape)


def _mm_ae(a, b):
    m, k = a.shape
    _, n = b.shape
    bm = 1000
    return pl.pallas_call(
        _mm_ae_kernel,
        grid=(m // bm,),
        in_specs=[
            pl.BlockSpec((bm, k), lambda i: (i, 0)),
            pl.BlockSpec((k, n), lambda i: (0, 0)),
        ],
        out_specs=[
            pl.BlockSpec((bm, n), lambda i: (i, 0)),
            pl.BlockSpec((8, n), lambda i: (0, 0)),
        ],
        out_shape=[
            jax.ShapeDtypeStruct((m, n), jnp.float32),
            jax.ShapeDtypeStruct((8, n), jnp.float32),
        ],
    )(a, b)


def _post_kernel(o_ref, h_ref, sw_ref, v_ref, out_ref):
    o = o_ref[...]
    v = v_ref[...]
    o = (o + v[0:1]) * v[1:2] + v[2:3]
    mu = jnp.mean(o, axis=1, keepdims=True)
    var = jnp.mean((o - mu) ** 2, axis=1, keepdims=True)
    o = (o - mu) / jnp.sqrt(var + 1e-5) * v[3:4] + v[4:5]
    o = o + jnp.dot(h_ref[...], sw_ref[...], preferred_element_type=jnp.float32) + v[5:6]
    out_ref[...] = jnp.where(o > 0, o, jnp.exp(o) - 1.0)


def _post(o, h, sw, vecs):
    m = o.shape[0]
    cin = h.shape[1]
    bm = 400
    return pl.pallas_call(
        _post_kernel,
        grid=(m // bm,),
        in_specs=[
            pl.BlockSpec((bm, 256), lambda i: (i, 0)),
            pl.BlockSpec((bm, cin), lambda i: (i, 0)),
            pl.BlockSpec((cin, 256), lambda i: (0, 0)),
            pl.BlockSpec((8, 256), lambda i: (0, 0)),
        ],
        out_specs=pl.BlockSpec((bm, 256), lambda i: (i, 0)),
        out_shape=jax.ShapeDtypeStruct((m, 256), jnp.float32),
    )(o, h, sw, vecs)


def _pool_kernel(b_ref, h_ref, out_ref):
    i = pl.program_id(0)

    @pl.when(i == 0)
    def _():
        out_ref[...] = jnp.zeros_like(out_ref)

    b = b_ref[0, 0, :]
    oh = (b[:, None] == lax.broadcasted_iota(jnp.int32, (1, G), 1)).astype(jnp.float32)
    hb = jnp.concatenate(
        [h_ref[...], jnp.ones((h_ref.shape[0], 128), jnp.float32)], axis=1)
    out_ref[...] += lax.dot_general(
        oh, hb, (((0,), (0,)), ((), ())), preferred_element_type=jnp.float32)

    @pl.when(i == pl.num_programs(0) - 1)
    def _():
        acc = out_ref[...]
        out_ref[...] = acc / jnp.maximum(acc[:, 256:257], 1.0)


def _pool(batch3, h):
    bm = 1000
    return pl.pallas_call(
        _pool_kernel,
        grid=(N // bm,),
        in_specs=[
            pl.BlockSpec((1, 1, bm), lambda i: (i, 0, 0)),
            pl.BlockSpec((bm, 256), lambda i: (i, 0)),
        ],
        out_specs=pl.BlockSpec((G, 384), lambda i: (0, 0)),
        out_shape=jax.ShapeDtypeStruct((G, 384), jnp.float32),
    )(batch3, h)


def _make_sc_gat(C, LOFF):
    mesh = plsc.VectorSubcoreMesh(core_axis_name="c", subcore_axis_name="s")
    NC = 2

    @functools.partial(
        pl.kernel,
        out_type=jax.ShapeDtypeStruct((NP, 256), jnp.float32),
        mesh=mesh,
        scratch_types=[
            pltpu.VMEM((R, 256), jnp.float32),
            pltpu.VMEM((40, 128), jnp.float32),
            pltpu.VMEM((R, 16), jnp.float32),
            pltpu.VMEM((3 * SCB * BE + 16,), jnp.int32),
            pltpu.VMEM((BE, 128), jnp.float32),
            pltpu.VMEM((BE, 128), jnp.float32),
            pltpu.VMEM((BE, 128), jnp.float32),
            pltpu.VMEM((BE, 128), jnp.float32),
            pltpu.VMEM((BE, 256), jnp.float32),
            pltpu.VMEM((BE, 256), jnp.float32),
            pltpu.SemaphoreType.DMA,
            pltpu.SemaphoreType.DMA,
        ],
    )
    def k(xw, asrc, adstc, aedge, src2, dstl2, eid2, startsb, out,
          acc, adst_loc, rden, idx_c,
          asrc_g0, asrc_g1, aedge_g0, aedge_g1, xw_g0, xw_g1,
          sem0, sem1):
        SB = SCB * BE
        src_c = idx_c.at[pl.ds(0, SB)]
        eid_c = idx_c.at[pl.ds(2 * SB, SB)]
        w = lax.axis_index("s") * NC + lax.axis_index("c")

        def adst_body(ac, _c):
            ao = pl.multiple_of(ac * 8, 8)
            pltpu.sync_copy(adstc.at[pl.ds(w * 40 + ao, 8)],
                            adst_loc.at[pl.ds(ao, 8)])
            return 0

        lax.fori_loop(0, 5, adst_body, 0)

        def half_body(half, _h):
            bkt = w * 2 + half
            rlo = bkt * R
            half20 = half * 20
            pltpu.sync_copy(startsb, idx_c.at[pl.ds(0, 80)])
            bvec = idx_c[pl.ds(bkt, 16)]
            b0 = bvec[0]
            b1 = bvec[1]

            def zero_body(n, _):
                zz = jnp.zeros((16,), jnp.float32)
                for kk in range(16):
                    acc[n, pl.ds(kk * 16, 16)] = zz
                rden[n, :] = zz
                return 0

            lax.fori_loop(0, R, zero_body, 0)

            def run(pass2):
                gbufs = [(asrc_g0, aedge_g0, xw_g0, sem0),
                         (asrc_g1, aedge_g1, xw_g1, sem1)]

                def issue(j, bufs):
                    a_g, e_g, x_g, sem = bufs
                    sidx = src_c.at[pl.ds(j * BE, BE)]
                    eidx = eid_c.at[pl.ds(j * BE, BE)]
                    pltpu.async_copy(asrc.at[sidx], a_g, sem)
                    pltpu.async_copy(aedge.at[eidx], e_g, sem)
                    if pass2:
                        pltpu.async_copy(xw.at[sidx], x_g, sem)

                def drain(bufs):
                    a_g, e_g, x_g, sem = bufs
                    didx = src_c.at[pl.ds(0, BE)]
                    pltpu.make_async_copy(asrc.at[didx], a_g, sem).wait()
                    pltpu.make_async_copy(aedge.at[didx], e_g, sem).wait()
                    if pass2:
                        pltpu.make_async_copy(xw.at[didx], x_g, sem).wait()

                def compute(j, bufs):
                    a_g, e_g, x_g, _ = bufs
                    dvec = idx_c[pl.ds(SB + j * BE, 16)]
                    for t in range(BE):
                        dstl = dvec[t]
                        al = (a_g[t, pl.ds(0, 16)] + e_g[t, pl.ds(LOFF, 16)]
                              + adst_loc[half20 + dstl // 8,
                                         pl.ds((dstl % 8) * 16, 16)])
                        al = jnp.maximum(al, al * 0.2)
                        ex = jnp.exp(al)
                        if not pass2:
                            plsc.addupdate(rden.at[dstl], ex)
                        else:
                            att = ex * rden[dstl, :]
                            for kk in range(16):
                                hk = (kk * 16) // C
                                plsc.addupdate(
                                    acc.at[dstl, pl.ds(kk * 16, 16)],
                                    x_g[t, pl.ds(kk * 16, 16)] * att[hk])

                def sc_body(sci, _c):
                    bs = pl.multiple_of(b0 + sci * SCB, 8)
                    cnt = jnp.minimum(b1 - bs, SCB)
                    be = pl.multiple_of(bs * BE, 128)
                    pltpu.sync_copy(src2.at[pl.ds(be, SB)],
                                    src_c.at[pl.ds(0, SB)])
                    pltpu.sync_copy(dstl2.at[pl.ds(be, SB)],
                                    idx_c.at[pl.ds(SB, SB)])
                    pltpu.sync_copy(eid2.at[pl.ds(be, SB)],
                                    eid_c.at[pl.ds(0, SB)])
                    issue(0, gbufs[0])

                    def blk(j, _b):
                        even = (j % 2) == 0

                        @pl.when((j + 1 < cnt) & even)
                        def _():
                            issue(j + 1, gbufs[1])

                        @pl.when((j + 1 < cnt) & jnp.logical_not(even))
                        def _():
                            issue(j + 1, gbufs[0])

                        @pl.when(even)
                        def _():
                            drain(gbufs[0])
                            compute(j, gbufs[0])

                        @pl.when(jnp.logical_not(even))
                        def _():
                            drain(gbufs[1])
                            compute(j, gbufs[1])

                        return 0

                    lax.fori_loop(0, cnt, blk, 0)
                    return 0

                nsc = (b1 - b0 + SCB - 1) // SCB
                lax.fori_loop(0, nsc, sc_body, 0)

            run(False)

            def rcp_body(n, _c):
                rden[n, :] = 1.0 / (rden[n, :] + 1e-16)
                return 0

            lax.fori_loop(0, R, rcp_body, 0)
            run(True)

            def out_body(oc, _c):
                oo = pl.multiple_of(rlo + oc * 8, 8)
                ol = pl.multiple_of(oc * 8, 8)
                pltpu.sync_copy(acc.at[pl.ds(ol, 8), pl.ds(0, 128)],
                                out.at[pl.ds(oo, 8), pl.ds(0, 128)])
                pltpu.sync_copy(acc.at[pl.ds(ol, 8), pl.ds(128, 128)],
                                out.at[pl.ds(oo, 8), pl.ds(128, 128)])
                return 0

            lax.fori_loop(0, 20, out_body, 0)
            return 0

        lax.fori_loop(0, 2, half_body, 0)

    return k


_sc_gat = [_make_sc_gat(32, 0), _make_sc_gat(32, 16),
           _make_sc_gat(32, 32), _make_sc_gat(256, 48)]


def kernel(x, edge_index, edge_attr, batch, params):
    wcat, skw, vecs, cdim = [], [], [], []
    for i, (cin, h, c, concat) in enumerate(CONV_CFG):
        lw = params[f'conv{i}_lin_w']
        bsrc = (lw.reshape(h, c, cin) * params[f'conv{i}_att_src'][:, :, None]).sum(1).T
        bdst = (lw.reshape(h, c, cin) * params[f'conv{i}_att_dst'][:, :, None]).sum(1).T
        w = jnp.zeros((cin, 512), jnp.float32)
        w = w.at[:, :256].set(lw.T)
        w = w.at[:, 256:256 + h].set(bsrc)
        w = w.at[:, 272:272 + h].set(bdst)
        wcat.append(w)
        if i > 0:
            skw.append(params[f'skip{i}_w'].T)
            skb = params[f'skip{i}_b']
        else:
            skw.append(jnp.zeros((cin, 256), jnp.float32))
            skb = jnp.zeros((256,), jnp.float32)
        bn_scale = params[f'bn{i}_g'] / np.sqrt(1.0 + 1e-5)
        v = jnp.stack([params[f'conv{i}_bias'], bn_scale, params[f'bn{i}_b'],
                       params[f'ln{i}_g'], params[f'ln{i}_b'], skb,
                       jnp.zeros((256,), jnp.float32), jnp.zeros((256,), jnp.float32)])
        vecs.append(v)
        cdim.append(c)

    bedge = jnp.zeros((D, 64), jnp.float32)
    for i, (cin, h, c, concat) in enumerate(CONV_CFG):
        lew = params[f'conv{i}_lin_edge_w']
        aev = params[f'conv{i}_att_edge']
        col = (lew.reshape(h, c, D) * aev[:, :, None]).sum(1).T
        bedge = bedge.at[:, 16 * i:16 * i + h].set(col)

    loop_idx = jnp.arange(N, dtype=jnp.int32)
    src = jnp.concatenate([edge_index[0], loop_idx])
    dst = jnp.concatenate([edge_index[1], loop_idx])
    eid = jnp.concatenate([jnp.arange(E, dtype=jnp.int32),
                           jnp.full((N,), E, jnp.int32)])
    order = jnp.argsort(dst)
    src_s, dst_s, eid_s = src[order], dst[order], eid[order]
    bucket_s = dst_s // R
    bnd = jnp.searchsorted(dst_s, jnp.arange(NB + 1, dtype=jnp.int32) * R).astype(jnp.int32)
    sizes = jnp.diff(bnd)
    astart = jnp.concatenate([
        jnp.zeros((1,), jnp.int32),
        jnp.cumsum(((sizes + ALIGN - 1) // ALIGN) * ALIGN).astype(jnp.int32)])
    pos = astart[bucket_s] + (jnp.arange(ETOT, dtype=jnp.int32) - bnd[bucket_s])
    fill = jnp.broadcast_to(jnp.array([0, 0, E + 1], jnp.int32), (EP, 3))
    packed = fill.at[pos].set(
        jnp.stack([src_s, dst_s - bucket_s * R, eid_s], axis=1))
    src2 = packed[:, 0]
    dstl2 = packed[:, 1]
    eid2 = packed[:, 2]
    startsb = jnp.pad(astart // BE, (0, 15))

    ae_all, ae_sum = _mm_ae(edge_attr, bedge)
    ae_mean = ae_sum[0:1] / E
    aedge_cmb = jnp.zeros((EA, 128), jnp.float32)
    aedge_cmb = aedge_cmb.at[:E, :64].set(ae_all)
    aedge_cmb = aedge_cmb.at[E:E + 1, :64].set(ae_mean)
    aedge_cmb = aedge_cmb.at[E + 1, :].set(-1e30)

    h_cur = x
    for i, (cin, h, c, concat) in enumerate(CONV_CFG):
        cat = _mm(h_cur, wcat[i], 400)
        xw = cat[:, :256]
        ad_all = jnp.pad(cat[:, 256:384], ((0, NP - N), (0, 0)))
        adst_cmp = jnp.pad(cat[:, 272:288], ((0, NP - N), (0, 0))).reshape(NP // 8, 128)
        out_sc = _sc_gat[i](xw, ad_all, adst_cmp, aedge_cmb,
                            src2, dstl2, eid2, startsb)[:N]
        h_cur = _post(out_sc, h_cur, skw[i], vecs[i])

    batch3 = batch.reshape(10, 1, 1000)
    return _pool(batch3, h_cur)[:, :256]
